# Initial kernel scaffold; baseline (speedup 1.0000x reference)
import jax
import jax.numpy as jnp
from jax.experimental import pallas as pl


def kernel(y):
    raise NotImplementedError("write your pallas kernel here")


# speedup vs baseline: 2.0410x; 2.0410x over previous
import jax
import jax.numpy as jnp
from jax import lax
from jax.experimental import pallas as pl
from jax.experimental.pallas import tpu as pltpu

N_CLASSES = 1000
ROWS_PER_BLOCK = 512


def _onehot_body(y_ref, out_ref):
    labels = y_ref[:, :]
    cols = lax.broadcasted_iota(jnp.int32, (ROWS_PER_BLOCK, N_CLASSES), 1)
    out_ref[:, :] = (cols == labels).astype(jnp.float32)


def kernel(y):
    n = y.shape[0] * y.shape[1]
    y_flat = y.reshape(n, 1)
    grid = n // ROWS_PER_BLOCK
    out = pl.pallas_call(
        _onehot_body,
        grid=(grid,),
        in_specs=[pl.BlockSpec((ROWS_PER_BLOCK, 1), lambda i: (i, 0))],
        out_specs=pl.BlockSpec((ROWS_PER_BLOCK, N_CLASSES), lambda i: (i, 0)),
        out_shape=jax.ShapeDtypeStruct((n, N_CLASSES), jnp.float32),
        compiler_params=pltpu.CompilerParams(
            dimension_semantics=("parallel",),
        ),
    )(y_flat)
    return out.reshape(y.shape + (N_CLASSES,))


# speedup vs baseline: 2.9043x; 1.4230x over previous
import jax
import jax.numpy as jnp
from jax import lax
from jax.experimental import pallas as pl
from jax.experimental.pallas import tpu as pltpu

N_CLASSES = 1000
B0 = 64


def _onehot_body(y_ref, out_ref):
    labels = y_ref[...]
    cols = lax.broadcasted_iota(jnp.int32, (B0, 26, N_CLASSES), 2)
    out_ref[...] = (cols == labels).astype(jnp.float32)


def kernel(y):
    b = y.shape[0]
    y3 = y.reshape(b, y.shape[1], 1)
    out = pl.pallas_call(
        _onehot_body,
        grid=(b // B0,),
        in_specs=[pl.BlockSpec((B0, 26, 1), lambda i: (i, 0, 0))],
        out_specs=pl.BlockSpec((B0, 26, N_CLASSES), lambda i: (i, 0, 0)),
        out_shape=jax.ShapeDtypeStruct((b, 26, N_CLASSES), jnp.float32),
        compiler_params=pltpu.CompilerParams(
            dimension_semantics=("parallel",),
        ),
    )(y3)
    return out
